# Initial kernel scaffold; baseline (speedup 1.0000x reference)
import jax
import jax.numpy as jnp
from jax.experimental import pallas as pl


def kernel(node_idx, edge_idx, node_table, edge_table):
    raise NotImplementedError("write your pallas kernel here")


# speedup vs baseline: 1.0445x; 1.0445x over previous
import jax
import jax.numpy as jnp
from jax.experimental import pallas as pl
from jax.experimental.pallas import tpu as pltpu
from jax.experimental.pallas import tpu_sc as plsc

_W = 128
_SPLIT = 2


def _double_idx(idx, split):
    return (split * idx.reshape(-1, 1) + jnp.arange(split, dtype=idx.dtype)).reshape(1, -1)


def kernel(node_idx, edge_idx, node_table, edge_table):
    B, S = node_idx.shape
    D = node_table.shape[1]
    N = B * S
    Dh = D // _SPLIT
    n_idx = _double_idx(node_idx, _SPLIT)
    e_idx = _double_idx(edge_idx, _SPLIT)
    nt = node_table.reshape(-1, Dh)
    et = edge_table.reshape(-1, Dh)
    M = _SPLIT * N

    mesh = plsc.VectorSubcoreMesh(
        core_axis_name="core", subcore_axis_name="subcore"
    )

    @pl.kernel(
        out_type=(
            jax.ShapeDtypeStruct((M, Dh), jnp.float32),
            jax.ShapeDtypeStruct((M, Dh), jnp.float32),
        ),
        mesh=mesh,
    )
    def run(nt_hbm, et_hbm, ni_hbm, ei_hbm, no_hbm, eo_hbm):
        def gather_body(tab_hbm):
            def body(i_vmem, o_vmem):
                pltpu.sync_copy(tab_hbm.at[i_vmem.at[0]], o_vmem)

            return body

        for tab, ih, oh in (
            (nt_hbm, ni_hbm, no_hbm),
            (et_hbm, ei_hbm, eo_hbm),
        ):
            pltpu.emit_pipeline(
                gather_body(tab),
                grid=(M // _W,),
                in_specs=[pl.BlockSpec((1, _W), index_map=lambda i: (0, i))],
                out_specs=[pl.BlockSpec((_W, Dh), index_map=lambda i: (i, 0))],
                core_axis_name=("core", "subcore"),
                dimension_semantics=(pltpu.PARALLEL,),
            )(ih, oh)

    node_out, edge_out = run(nt, et, n_idx, e_idx)
    return node_out.reshape(B, S, D), edge_out.reshape(B, S, D)


# speedup vs baseline: 2.3633x; 2.2627x over previous
import jax
import jax.numpy as jnp
from jax.experimental import pallas as pl
from jax.experimental.pallas import tpu as pltpu
from jax.experimental.pallas import tpu_sc as plsc

_W = 128
_SPLIT = 2


def _half_idx(idx_t, split):
    return (
        split * idx_t.reshape(1, -1)
        + jnp.arange(split, dtype=idx_t.dtype).reshape(-1, 1)
    )


def kernel(node_idx, edge_idx, node_table, edge_table):
    B, S = node_idx.shape
    D = node_table.shape[1]
    N = B * S
    Dh = D // _SPLIT
    n_idx = _half_idx(node_idx.T.reshape(-1), _SPLIT)
    e_idx = _half_idx(edge_idx.T.reshape(-1), _SPLIT)
    nt = node_table.reshape(-1, Dh)
    et = edge_table.reshape(-1, Dh)

    mesh = plsc.VectorSubcoreMesh(
        core_axis_name="core", subcore_axis_name="subcore"
    )

    @pl.kernel(
        out_type=(
            jax.ShapeDtypeStruct((N, D), jnp.float32),
            jax.ShapeDtypeStruct((N, D), jnp.float32),
        ),
        mesh=mesh,
    )
    def run(nt_hbm, et_hbm, ni_hbm, ei_hbm, no_hbm, eo_hbm):
        def gather_body(tab_hbm):
            def body(i_vmem, o_vmem):
                pltpu.sync_copy(tab_hbm.at[i_vmem.at[0]], o_vmem)

            return body

        for tab, ih, oh in (
            (nt_hbm, ni_hbm, no_hbm),
            (et_hbm, ei_hbm, eo_hbm),
        ):
            pltpu.emit_pipeline(
                gather_body(tab),
                grid=(N // _W, _SPLIT),
                in_specs=[
                    pl.BlockSpec((1, _W), index_map=lambda i, j: (j, i))
                ],
                out_specs=[
                    pl.BlockSpec((_W, Dh), index_map=lambda i, j: (i, j))
                ],
                core_axis_name=("core", "subcore"),
                dimension_semantics=(pltpu.PARALLEL, pltpu.PARALLEL),
            )(ih, oh)

    node_out, edge_out = run(nt, et, n_idx, e_idx)
    node_out = node_out.reshape(S, B, D).transpose(1, 0, 2)
    edge_out = edge_out.reshape(S, B, D).transpose(1, 0, 2)
    return node_out, edge_out


# speedup vs baseline: 3.8018x; 1.6087x over previous
import jax
import jax.numpy as jnp
from jax.experimental import pallas as pl
from jax.experimental.pallas import tpu as pltpu
from jax.experimental.pallas import tpu_sc as plsc

_W = 128
_SPLIT = 2

_TC_ROWS = 256
_TC_V = 128


def _sc_lookup(idx_t, table):
    N = idx_t.shape[0]
    D = table.shape[1]
    Dh = D // _SPLIT
    idx2 = (
        _SPLIT * idx_t.reshape(1, -1)
        + jnp.arange(_SPLIT, dtype=idx_t.dtype).reshape(-1, 1)
    )
    tab = table.reshape(-1, Dh)

    mesh = plsc.VectorSubcoreMesh(
        core_axis_name="core", subcore_axis_name="subcore"
    )

    @pl.kernel(
        out_type=jax.ShapeDtypeStruct((N, D), jnp.float32),
        mesh=mesh,
    )
    def run(tab_hbm, idx_hbm, out_hbm):
        def body(i_vmem, o_vmem):
            pltpu.sync_copy(tab_hbm.at[i_vmem.at[0]], o_vmem)

        pltpu.emit_pipeline(
            body,
            grid=(N // _W, _SPLIT),
            in_specs=[pl.BlockSpec((1, _W), index_map=lambda i, j: (j, i))],
            out_specs=[pl.BlockSpec((_W, Dh), index_map=lambda i, j: (i, j))],
            core_axis_name=("core", "subcore"),
            dimension_semantics=(pltpu.PARALLEL, pltpu.PARALLEL),
        )(idx_hbm, out_hbm)

    return run(tab, idx2)


def _tc_lookup(idx_t, table):
    N = idx_t.shape[0]
    D = table.shape[1]
    V = table.shape[0]
    steps = N // _TC_ROWS
    idx_cols = idx_t.reshape(steps, _TC_ROWS, 1)
    tab_pad = jnp.zeros((_TC_V, D), jnp.float32).at[:V].set(table)
    hi = tab_pad.astype(jnp.bfloat16)
    lo = (tab_pad - hi.astype(jnp.float32)).astype(jnp.bfloat16)

    def body(i_ref, hi_ref, lo_ref, o_ref):
        onehot = (
            i_ref[0] == jax.lax.broadcasted_iota(jnp.int32, (_TC_ROWS, _TC_V), 1)
        ).astype(jnp.bfloat16)
        acc = jax.lax.dot(onehot, hi_ref[...], preferred_element_type=jnp.float32)
        acc += jax.lax.dot(onehot, lo_ref[...], preferred_element_type=jnp.float32)
        o_ref[...] = acc

    return pl.pallas_call(
        body,
        grid=(steps,),
        in_specs=[
            pl.BlockSpec((1, _TC_ROWS, 1), lambda i: (i, 0, 0)),
            pl.BlockSpec((_TC_V, D), lambda i: (0, 0)),
            pl.BlockSpec((_TC_V, D), lambda i: (0, 0)),
        ],
        out_specs=pl.BlockSpec((_TC_ROWS, D), lambda i: (i, 0)),
        out_shape=jax.ShapeDtypeStruct((N, D), jnp.float32),
    )(idx_cols, hi, lo)


def kernel(node_idx, edge_idx, node_table, edge_table):
    B, S = node_idx.shape
    D = node_table.shape[1]
    node_out = _sc_lookup(node_idx.T.reshape(-1), node_table)
    edge_out = _tc_lookup(edge_idx.T.reshape(-1), edge_table)
    node_out = node_out.reshape(S, B, D).transpose(1, 0, 2)
    edge_out = edge_out.reshape(S, B, D).transpose(1, 0, 2)
    return node_out, edge_out
